# baseline (speedup 1.0000x reference)
import functools

import jax
import jax.numpy as jnp
from jax import lax
from jax.experimental import pallas as pl
from jax.experimental.pallas import tpu as pltpu
from jax.experimental.pallas import tpu_sc as plsc

_B = 16384
_D = 64
_K = 20
_N = 1000000
_NC = 2
_NS = 16
_NW = _NC * _NS
_BPW = _B // _NW
_CB = 16
_NCHUNK = _BPW // _CB
_IDXW = 64
_NGROWS = _BPW * _K // _IDXW
_ROWS_PER_CHUNK = _CB * _K // _IDXW

_XB = 512
_XSTEPS = (_N + _XB - 1) // _XB
_WROWS = _XSTEPS * _XB


def _pack_body(vt_ref, ut_ref, w_ref):
    w_ref[...] = jnp.concatenate(
        [jnp.transpose(vt_ref[...]), jnp.transpose(ut_ref[...])], axis=1)


def _pack_tables(vt, ut):
    return pl.pallas_call(
        _pack_body,
        grid=(_XSTEPS,),
        in_specs=[pl.BlockSpec((_D, _XB), lambda s: (0, s)),
                  pl.BlockSpec((_D, _XB), lambda s: (0, s))],
        out_specs=pl.BlockSpec((_XB, 2 * _D), lambda s: (s, 0)),
        out_shape=jax.ShapeDtypeStruct((_WROWS, 2 * _D), jnp.float32),
    )(vt, ut)


def _sc_dots_body(c_hbm, o_hbm, ng_hbm, w_hbm, sp_hbm, sn_hbm,
                  c_v, o_v, ng_v, vc_b, uo_b, un_b, pt, pt_sp,
                  sp_res, sn_res, sem):
    wid = lax.axis_index("s") * _NC + lax.axis_index("c")
    base = wid * _BPW
    pltpu.sync_copy(c_hbm.at[pl.ds(base, _BPW)], c_v)
    pltpu.sync_copy(o_hbm.at[pl.ds(base, _BPW)], o_v)
    pltpu.sync_copy(ng_hbm.at[pl.ds(wid * _NGROWS, _NGROWS), :], ng_v)

    lane = lax.iota(jnp.int32, 16)
    cols = [jnp.full((16,), cc, jnp.int32) for cc in range(16)]

    def lane_sums(ptref, rows):
        acc = plsc.load_gather(ptref, [rows, cols[0]])
        for cc in range(1, 16):
            acc = acc + plsc.load_gather(ptref, [rows, cols[cc]])
        return acc

    def chunk(ch, carry):
        cp_vc = pltpu.async_copy(
            w_hbm.at[c_v.at[pl.ds(ch * _CB, _CB)]], vc_b, sem)
        cp_uo = pltpu.async_copy(
            w_hbm.at[o_v.at[pl.ds(ch * _CB, _CB)]], uo_b, sem)
        cps = []
        for r in range(_ROWS_PER_CHUNK):
            cps.append(pltpu.async_copy(
                w_hbm.at[ng_v.at[ch * _ROWS_PER_CHUNK + r]],
                un_b.at[pl.ds(r * _IDXW, _IDXW), :], sem))
        cp_vc.wait()
        cp_uo.wait()
        for cp in cps:
            cp.wait()

        for sb in range(4):
            vcreg = [[vc_b[sb * 4 + b, pl.ds(16 * j, 16)] for j in range(4)]
                     for b in range(4)]
            for b in range(4):
                part = vcreg[b][0] * uo_b[sb * 4 + b, pl.ds(64, 16)]
                for j in range(1, 4):
                    part = part + vcreg[b][j] * uo_b[sb * 4 + b,
                                                     pl.ds(64 + 16 * j, 16)]
                pt_sp[sb * 4 + b, pl.ds(0, 16)] = part
            for g in range(5):
                slot = g % 4
                for i in range(16):
                    q = g * 16 + i
                    p = sb * 80 + q
                    lb = q // _K
                    part = vcreg[lb][0] * un_b[p, pl.ds(64, 16)]
                    for j in range(1, 4):
                        part = part + vcreg[lb][j] * un_b[p,
                                                          pl.ds(64 + 16 * j,
                                                                16)]
                    pt[slot * 16 + i, pl.ds(0, 16)] = part
                rv = lane_sums(pt, slot * 16 + lane)
                sn_res[pl.ds(ch * _CB * _K + sb * 80 + g * 16, 16)] = rv
        sp_res[pl.ds(ch * _CB, _CB)] = lane_sums(pt_sp, lane)
        return carry

    lax.fori_loop(0, _NCHUNK, chunk, 0)

    pltpu.sync_copy(sp_res, sp_hbm.at[pl.ds(base, _BPW)])
    pltpu.sync_copy(sn_res, sn_hbm.at[pl.ds(wid * _BPW * _K, _BPW * _K)])


_sc_dots = functools.partial(
    pl.kernel,
    out_type=(jax.ShapeDtypeStruct((_B,), jnp.float32),
              jax.ShapeDtypeStruct((_B * _K,), jnp.float32)),
    mesh=plsc.VectorSubcoreMesh(core_axis_name="c", subcore_axis_name="s"),
    compiler_params=pltpu.CompilerParams(
        needs_layout_passes=False, use_tc_tiling_on_sc=True),
    scratch_types=[
        pltpu.VMEM((_BPW,), jnp.int32),
        pltpu.VMEM((_BPW,), jnp.int32),
        pltpu.VMEM((_NGROWS, _IDXW), jnp.int32),
        pltpu.VMEM((_CB, 2 * _D), jnp.float32),
        pltpu.VMEM((_CB, 2 * _D), jnp.float32),
        pltpu.VMEM((_CB * _K, 2 * _D), jnp.float32),
        pltpu.VMEM((64, 17), jnp.float32),
        pltpu.VMEM((16, 17), jnp.float32),
        pltpu.VMEM((_BPW,), jnp.float32),
        pltpu.VMEM((_BPW * _K,), jnp.float32),
        pltpu.SemaphoreType.DMA,
    ],
)(_sc_dots_body)


def _logsig(x):
    return jnp.minimum(x, 0.0) - jnp.log1p(jnp.exp(-jnp.abs(x)))


def _loss_body(sp_ref, sn_ref, out_ref):
    lp = _logsig(sp_ref[...])
    ln = _logsig(-sn_ref[...])
    out_ref[...] = jnp.reshape(-(jnp.sum(lp) + jnp.sum(ln)) / _B, (1, 1))


def kernel(c, o, ng, V, U):
    ng2 = ng.reshape(_B * _K // _IDXW, _IDXW)
    w = _pack_tables(V.T, U.T)
    sp, sn = _sc_dots(c, o, ng2, w)
    loss = pl.pallas_call(
        _loss_body,
        out_shape=jax.ShapeDtypeStruct((1, 1), jnp.float32),
    )(sp.reshape(128, 128), sn.reshape(_B * _K // 128, 128))
    return loss[0, 0]


# speedup vs baseline: 1.8084x; 1.8084x over previous
import functools

import jax
import jax.numpy as jnp
from jax import lax
from jax.experimental import pallas as pl
from jax.experimental.pallas import tpu as pltpu
from jax.experimental.pallas import tpu_sc as plsc

_B = 16384
_D = 64
_K = 20
_N = 1000000
_NC = 2
_NS = 16
_NW = _NC * _NS
_BPW = _B // _NW
_CB = 16
_NCHUNK = _BPW // _CB
_IDXW = 64
_NGROWS = _BPW * _K // _IDXW
_ROWS_PER_CHUNK = _CB * _K // _IDXW

_XB = 2048
_XSTEPS = (_N + _XB - 1) // _XB
_WROWS = _XSTEPS * _XB


def _pack_body(vt_ref, ut_ref, w_ref):
    w_ref[...] = jnp.concatenate(
        [jnp.transpose(vt_ref[...]), jnp.transpose(ut_ref[...])], axis=1)


def _pack_tables(vt, ut):
    return pl.pallas_call(
        _pack_body,
        grid=(_XSTEPS,),
        in_specs=[pl.BlockSpec((_D, _XB), lambda s: (0, s)),
                  pl.BlockSpec((_D, _XB), lambda s: (0, s))],
        out_specs=pl.BlockSpec((_XB, 2 * _D), lambda s: (s, 0)),
        out_shape=jax.ShapeDtypeStruct((_WROWS, 2 * _D), jnp.float32),
    )(vt, ut)


def _sc_dots_body(c_hbm, o_hbm, ng_hbm, w_hbm, sp_hbm, sn_hbm,
                  c_v, o_v, ng_v, vc_b, uo_b, un_b, pt, pt_sp,
                  sp_res, sn_res, sem):
    wid = lax.axis_index("s") * _NC + lax.axis_index("c")
    base = wid * _BPW
    pltpu.sync_copy(c_hbm.at[pl.ds(base, _BPW)], c_v)
    pltpu.sync_copy(o_hbm.at[pl.ds(base, _BPW)], o_v)
    pltpu.sync_copy(ng_hbm.at[pl.ds(wid * _NGROWS, _NGROWS), :], ng_v)

    lane = lax.iota(jnp.int32, 16)
    cols = [jnp.full((16,), cc, jnp.int32) for cc in range(16)]

    def lane_sums(ptref, rows):
        acc = plsc.load_gather(ptref, [rows, cols[0]])
        for cc in range(1, 16):
            acc = acc + plsc.load_gather(ptref, [rows, cols[cc]])
        return acc

    def chunk(ch, carry):
        cp_vc = pltpu.async_copy(
            w_hbm.at[c_v.at[pl.ds(ch * _CB, _CB)]], vc_b, sem)
        cp_uo = pltpu.async_copy(
            w_hbm.at[o_v.at[pl.ds(ch * _CB, _CB)]], uo_b, sem)
        cps = []
        for r in range(_ROWS_PER_CHUNK):
            cps.append(pltpu.async_copy(
                w_hbm.at[ng_v.at[ch * _ROWS_PER_CHUNK + r]],
                un_b.at[pl.ds(r * _IDXW, _IDXW), :], sem))
        cp_vc.wait()
        cp_uo.wait()
        for cp in cps:
            cp.wait()

        for sb in range(4):
            vcreg = [[vc_b[sb * 4 + b, pl.ds(16 * j, 16)] for j in range(4)]
                     for b in range(4)]
            for b in range(4):
                part = vcreg[b][0] * uo_b[sb * 4 + b, pl.ds(64, 16)]
                for j in range(1, 4):
                    part = part + vcreg[b][j] * uo_b[sb * 4 + b,
                                                     pl.ds(64 + 16 * j, 16)]
                pt_sp[sb * 4 + b, pl.ds(0, 16)] = part
            for g in range(5):
                slot = g % 4
                for i in range(16):
                    q = g * 16 + i
                    p = sb * 80 + q
                    lb = q // _K
                    part = vcreg[lb][0] * un_b[p, pl.ds(64, 16)]
                    for j in range(1, 4):
                        part = part + vcreg[lb][j] * un_b[p,
                                                          pl.ds(64 + 16 * j,
                                                                16)]
                    pt[slot * 16 + i, pl.ds(0, 16)] = part
                rv = lane_sums(pt, slot * 16 + lane)
                sn_res[pl.ds(ch * _CB * _K + sb * 80 + g * 16, 16)] = rv
        sp_res[pl.ds(ch * _CB, _CB)] = lane_sums(pt_sp, lane)
        return carry

    lax.fori_loop(0, _NCHUNK, chunk, 0)

    pltpu.sync_copy(sp_res, sp_hbm.at[pl.ds(base, _BPW)])
    pltpu.sync_copy(sn_res, sn_hbm.at[pl.ds(wid * _BPW * _K, _BPW * _K)])


_sc_dots = functools.partial(
    pl.kernel,
    out_type=(jax.ShapeDtypeStruct((_B,), jnp.float32),
              jax.ShapeDtypeStruct((_B * _K,), jnp.float32)),
    mesh=plsc.VectorSubcoreMesh(core_axis_name="c", subcore_axis_name="s"),
    compiler_params=pltpu.CompilerParams(
        needs_layout_passes=False, use_tc_tiling_on_sc=True),
    scratch_types=[
        pltpu.VMEM((_BPW,), jnp.int32),
        pltpu.VMEM((_BPW,), jnp.int32),
        pltpu.VMEM((_NGROWS, _IDXW), jnp.int32),
        pltpu.VMEM((_CB, 2 * _D), jnp.float32),
        pltpu.VMEM((_CB, 2 * _D), jnp.float32),
        pltpu.VMEM((_CB * _K, 2 * _D), jnp.float32),
        pltpu.VMEM((64, 17), jnp.float32),
        pltpu.VMEM((16, 17), jnp.float32),
        pltpu.VMEM((_BPW,), jnp.float32),
        pltpu.VMEM((_BPW * _K,), jnp.float32),
        pltpu.SemaphoreType.DMA,
    ],
)(_sc_dots_body)


def _logsig(x):
    return jnp.minimum(x, 0.0) - jnp.log1p(jnp.exp(-jnp.abs(x)))


def _loss_body(sp_ref, sn_ref, out_ref):
    lp = _logsig(sp_ref[...])
    ln = _logsig(-sn_ref[...])
    out_ref[...] = jnp.reshape(-(jnp.sum(lp) + jnp.sum(ln)) / _B, (1, 1))


def kernel(c, o, ng, V, U):
    ng2 = ng.reshape(_B * _K // _IDXW, _IDXW)
    w = _pack_tables(V.T, U.T)
    sp, sn = _sc_dots(c, o, ng2, w)
    loss = pl.pallas_call(
        _loss_body,
        out_shape=jax.ShapeDtypeStruct((1, 1), jnp.float32),
    )(sp.reshape(128, 128), sn.reshape(_B * _K // 128, 128))
    return loss[0, 0]


# speedup vs baseline: 2.3232x; 1.2847x over previous
import functools

import jax
import jax.numpy as jnp
from jax import lax
from jax.experimental import pallas as pl
from jax.experimental.pallas import tpu as pltpu
from jax.experimental.pallas import tpu_sc as plsc

_B = 16384
_D = 64
_K = 20
_N = 1000000
_NC = 2
_NS = 16
_NW = _NC * _NS
_BPW = _B // _NW
_CB = 16
_NCHUNK = _BPW // _CB
_IDXW = 64
_NGROWS = _BPW * _K // _IDXW
_ROWS_PER_CHUNK = _CB * _K // _IDXW

_XB = 8192
_XSTEPS = (_N + _XB - 1) // _XB
_WROWS = _XSTEPS * _XB


def _pack_body(vt_ref, ut_ref, w_ref):
    w_ref[...] = jnp.concatenate(
        [jnp.transpose(vt_ref[...]), jnp.transpose(ut_ref[...])], axis=1)


def _pack_tables(vt, ut):
    return pl.pallas_call(
        _pack_body,
        grid=(_XSTEPS,),
        in_specs=[pl.BlockSpec((_D, _XB), lambda s: (0, s)),
                  pl.BlockSpec((_D, _XB), lambda s: (0, s))],
        out_specs=pl.BlockSpec((_XB, 2 * _D), lambda s: (s, 0)),
        out_shape=jax.ShapeDtypeStruct((_WROWS, 2 * _D), jnp.float32),
    )(vt, ut)


def _sc_dots_body(c_hbm, o_hbm, ng_hbm, w_hbm, sp_hbm, sn_hbm,
                  c_v, o_v, ng_v, vc_b, uo_b, un_b, pt, pt_sp,
                  sp_res, sn_res, sem):
    wid = lax.axis_index("s") * _NC + lax.axis_index("c")
    base = wid * _BPW
    pltpu.sync_copy(c_hbm.at[pl.ds(base, _BPW)], c_v)
    pltpu.sync_copy(o_hbm.at[pl.ds(base, _BPW)], o_v)
    pltpu.sync_copy(ng_hbm.at[pl.ds(wid * _NGROWS, _NGROWS), :], ng_v)

    lane = lax.iota(jnp.int32, 16)
    cols = [jnp.full((16,), cc, jnp.int32) for cc in range(16)]

    def lane_sums(ptref, rows):
        acc = plsc.load_gather(ptref, [rows, cols[0]])
        for cc in range(1, 16):
            acc = acc + plsc.load_gather(ptref, [rows, cols[cc]])
        return acc

    def chunk(ch, carry):
        cp_vc = pltpu.async_copy(
            w_hbm.at[c_v.at[pl.ds(ch * _CB, _CB)]], vc_b, sem)
        cp_uo = pltpu.async_copy(
            w_hbm.at[o_v.at[pl.ds(ch * _CB, _CB)]], uo_b, sem)
        cps = []
        for r in range(_ROWS_PER_CHUNK):
            cps.append(pltpu.async_copy(
                w_hbm.at[ng_v.at[ch * _ROWS_PER_CHUNK + r]],
                un_b.at[pl.ds(r * _IDXW, _IDXW), :], sem))
        cp_vc.wait()
        cp_uo.wait()
        for cp in cps:
            cp.wait()

        for sb in range(4):
            vcreg = [[vc_b[sb * 4 + b, pl.ds(16 * j, 16)] for j in range(4)]
                     for b in range(4)]
            for b in range(4):
                part = vcreg[b][0] * uo_b[sb * 4 + b, pl.ds(64, 16)]
                for j in range(1, 4):
                    part = part + vcreg[b][j] * uo_b[sb * 4 + b,
                                                     pl.ds(64 + 16 * j, 16)]
                pt_sp[sb * 4 + b, pl.ds(0, 16)] = part
            for g in range(5):
                slot = g % 4
                for i in range(16):
                    q = g * 16 + i
                    p = sb * 80 + q
                    lb = q // _K
                    part = vcreg[lb][0] * un_b[p, pl.ds(64, 16)]
                    for j in range(1, 4):
                        part = part + vcreg[lb][j] * un_b[p,
                                                          pl.ds(64 + 16 * j,
                                                                16)]
                    pt[slot * 16 + i, pl.ds(0, 16)] = part
                rv = lane_sums(pt, slot * 16 + lane)
                sn_res[pl.ds(ch * _CB * _K + sb * 80 + g * 16, 16)] = rv
        sp_res[pl.ds(ch * _CB, _CB)] = lane_sums(pt_sp, lane)
        return carry

    lax.fori_loop(0, _NCHUNK, chunk, 0)

    pltpu.sync_copy(sp_res, sp_hbm.at[pl.ds(base, _BPW)])
    pltpu.sync_copy(sn_res, sn_hbm.at[pl.ds(wid * _BPW * _K, _BPW * _K)])


_sc_dots = functools.partial(
    pl.kernel,
    out_type=(jax.ShapeDtypeStruct((_B,), jnp.float32),
              jax.ShapeDtypeStruct((_B * _K,), jnp.float32)),
    mesh=plsc.VectorSubcoreMesh(core_axis_name="c", subcore_axis_name="s"),
    compiler_params=pltpu.CompilerParams(
        needs_layout_passes=False, use_tc_tiling_on_sc=True),
    scratch_types=[
        pltpu.VMEM((_BPW,), jnp.int32),
        pltpu.VMEM((_BPW,), jnp.int32),
        pltpu.VMEM((_NGROWS, _IDXW), jnp.int32),
        pltpu.VMEM((_CB, 2 * _D), jnp.float32),
        pltpu.VMEM((_CB, 2 * _D), jnp.float32),
        pltpu.VMEM((_CB * _K, 2 * _D), jnp.float32),
        pltpu.VMEM((64, 17), jnp.float32),
        pltpu.VMEM((16, 17), jnp.float32),
        pltpu.VMEM((_BPW,), jnp.float32),
        pltpu.VMEM((_BPW * _K,), jnp.float32),
        pltpu.SemaphoreType.DMA,
    ],
)(_sc_dots_body)


def _logsig(x):
    return jnp.minimum(x, 0.0) - jnp.log1p(jnp.exp(-jnp.abs(x)))


def _loss_body(sp_ref, sn_ref, out_ref):
    lp = _logsig(sp_ref[...])
    ln = _logsig(-sn_ref[...])
    out_ref[...] = jnp.reshape(-(jnp.sum(lp) + jnp.sum(ln)) / _B, (1, 1))


def kernel(c, o, ng, V, U):
    ng2 = ng.reshape(_B * _K // _IDXW, _IDXW)
    w = _pack_tables(V.T, U.T)
    sp, sn = _sc_dots(c, o, ng2, w)
    loss = pl.pallas_call(
        _loss_body,
        out_shape=jax.ShapeDtypeStruct((1, 1), jnp.float32),
    )(sp.reshape(128, 128), sn.reshape(_B * _K // 128, 128))
    return loss[0, 0]


# speedup vs baseline: 2.4217x; 1.0424x over previous
import functools

import jax
import jax.numpy as jnp
from jax import lax
from jax.experimental import pallas as pl
from jax.experimental.pallas import tpu as pltpu
from jax.experimental.pallas import tpu_sc as plsc

_B = 16384
_D = 64
_K = 20
_N = 1000000
_NC = 2
_NS = 16
_NW = _NC * _NS
_BPW = _B // _NW
_CB = 16
_NCHUNK = _BPW // _CB
_IDXW = 64
_NGROWS = _BPW * _K // _IDXW
_ROWS_PER_CHUNK = _CB * _K // _IDXW

_XB = 16384
_XSTEPS = (_N + _XB - 1) // _XB
_WROWS = _XSTEPS * _XB


def _pack_body(vt_ref, ut_ref, w_ref):
    w_ref[...] = jnp.concatenate(
        [jnp.transpose(vt_ref[...]), jnp.transpose(ut_ref[...])], axis=1)


def _pack_tables(vt, ut):
    return pl.pallas_call(
        _pack_body,
        grid=(_XSTEPS,),
        in_specs=[pl.BlockSpec((_D, _XB), lambda s: (0, s)),
                  pl.BlockSpec((_D, _XB), lambda s: (0, s))],
        out_specs=pl.BlockSpec((_XB, 2 * _D), lambda s: (s, 0)),
        out_shape=jax.ShapeDtypeStruct((_WROWS, 2 * _D), jnp.float32),
    )(vt, ut)


def _sc_dots_body(c_hbm, o_hbm, ng_hbm, w_hbm, sp_hbm, sn_hbm,
                  c_v, o_v, ng_v, vc_b, uo_b, un_b, pt, pt_sp,
                  sp_res, sn_res, sem):
    wid = lax.axis_index("s") * _NC + lax.axis_index("c")
    base = wid * _BPW
    pltpu.sync_copy(c_hbm.at[pl.ds(base, _BPW)], c_v)
    pltpu.sync_copy(o_hbm.at[pl.ds(base, _BPW)], o_v)
    pltpu.sync_copy(ng_hbm.at[pl.ds(wid * _NGROWS, _NGROWS), :], ng_v)

    lane = lax.iota(jnp.int32, 16)
    cols = [jnp.full((16,), cc, jnp.int32) for cc in range(16)]

    def lane_sums(ptref, rows):
        acc = plsc.load_gather(ptref, [rows, cols[0]])
        for cc in range(1, 16):
            acc = acc + plsc.load_gather(ptref, [rows, cols[cc]])
        return acc

    def chunk(ch, carry):
        cp_vc = pltpu.async_copy(
            w_hbm.at[c_v.at[pl.ds(ch * _CB, _CB)]], vc_b, sem)
        cp_uo = pltpu.async_copy(
            w_hbm.at[o_v.at[pl.ds(ch * _CB, _CB)]], uo_b, sem)
        cps = []
        for r in range(_ROWS_PER_CHUNK):
            cps.append(pltpu.async_copy(
                w_hbm.at[ng_v.at[ch * _ROWS_PER_CHUNK + r]],
                un_b.at[pl.ds(r * _IDXW, _IDXW), :], sem))
        cp_vc.wait()
        cp_uo.wait()
        for cp in cps:
            cp.wait()

        for sb in range(4):
            vcreg = [[vc_b[sb * 4 + b, pl.ds(16 * j, 16)] for j in range(4)]
                     for b in range(4)]
            for b in range(4):
                part = vcreg[b][0] * uo_b[sb * 4 + b, pl.ds(64, 16)]
                for j in range(1, 4):
                    part = part + vcreg[b][j] * uo_b[sb * 4 + b,
                                                     pl.ds(64 + 16 * j, 16)]
                pt_sp[sb * 4 + b, pl.ds(0, 16)] = part
            for g in range(5):
                slot = g % 4
                for i in range(16):
                    q = g * 16 + i
                    p = sb * 80 + q
                    lb = q // _K
                    part = vcreg[lb][0] * un_b[p, pl.ds(64, 16)]
                    for j in range(1, 4):
                        part = part + vcreg[lb][j] * un_b[p,
                                                          pl.ds(64 + 16 * j,
                                                                16)]
                    pt[slot * 16 + i, pl.ds(0, 16)] = part
                rv = lane_sums(pt, slot * 16 + lane)
                sn_res[pl.ds(ch * _CB * _K + sb * 80 + g * 16, 16)] = rv
        sp_res[pl.ds(ch * _CB, _CB)] = lane_sums(pt_sp, lane)
        return carry

    lax.fori_loop(0, _NCHUNK, chunk, 0)

    pltpu.sync_copy(sp_res, sp_hbm.at[pl.ds(base, _BPW)])
    pltpu.sync_copy(sn_res, sn_hbm.at[pl.ds(wid * _BPW * _K, _BPW * _K)])


_sc_dots = functools.partial(
    pl.kernel,
    out_type=(jax.ShapeDtypeStruct((_B,), jnp.float32),
              jax.ShapeDtypeStruct((_B * _K,), jnp.float32)),
    mesh=plsc.VectorSubcoreMesh(core_axis_name="c", subcore_axis_name="s"),
    compiler_params=pltpu.CompilerParams(
        needs_layout_passes=False, use_tc_tiling_on_sc=True),
    scratch_types=[
        pltpu.VMEM((_BPW,), jnp.int32),
        pltpu.VMEM((_BPW,), jnp.int32),
        pltpu.VMEM((_NGROWS, _IDXW), jnp.int32),
        pltpu.VMEM((_CB, 2 * _D), jnp.float32),
        pltpu.VMEM((_CB, 2 * _D), jnp.float32),
        pltpu.VMEM((_CB * _K, 2 * _D), jnp.float32),
        pltpu.VMEM((64, 17), jnp.float32),
        pltpu.VMEM((16, 17), jnp.float32),
        pltpu.VMEM((_BPW,), jnp.float32),
        pltpu.VMEM((_BPW * _K,), jnp.float32),
        pltpu.SemaphoreType.DMA,
    ],
)(_sc_dots_body)


def _logsig(x):
    return jnp.minimum(x, 0.0) - jnp.log1p(jnp.exp(-jnp.abs(x)))


def _loss_body(sp_ref, sn_ref, out_ref):
    lp = _logsig(sp_ref[...])
    ln = _logsig(-sn_ref[...])
    out_ref[...] = jnp.reshape(-(jnp.sum(lp) + jnp.sum(ln)) / _B, (1, 1))


def kernel(c, o, ng, V, U):
    ng2 = ng.reshape(_B * _K // _IDXW, _IDXW)
    w = _pack_tables(V.T, U.T)
    sp, sn = _sc_dots(c, o, ng2, w)
    loss = pl.pallas_call(
        _loss_body,
        out_shape=jax.ShapeDtypeStruct((1, 1), jnp.float32),
    )(sp.reshape(128, 128), sn.reshape(_B * _K // 128, 128))
    return loss[0, 0]
